# baseline (speedup 1.0000x reference)
import jax
import jax.numpy as jnp
from jax import lax
from jax.experimental import pallas as pl
from jax.experimental.pallas import tpu as pltpu
from jax.experimental.pallas import tpu_sc as plsc

D = 64
N_NEG = 20
NDOT = N_NEG + 1
NC, NS = 2, 16
NW = NC * NS
CHUNK = 32


def _transpose_body(wit_hbm, wjt_hbm, taili_hbm, tailj_hbm, wio_hbm, wjo_hbm,
                    inbuf0, inbuf1, obuf0, obuf1, sem_in, sem_out):
  inbufs = (inbuf0, inbuf1)
  obufs = (obuf0, obuf1)
  V = wit_hbm.shape[1]
  n_full = V // 128
  tail = V - n_full * 128
  n_iters = (n_full + NW - 1) // NW

  wid = lax.axis_index("s") * NC + lax.axis_index("c")
  lanes = lax.iota(jnp.int32, 16)
  pbase = [(16 * k + lanes) * D for k in range(8)]

  @pl.when(wid == 0)
  def _():
    pltpu.sync_copy(taili_hbm, obuf0.at[pl.ds(0, tail * D)])
    pltpu.sync_copy(obuf0.at[pl.ds(0, tail * D)],
                    wio_hbm.at[pl.ds(n_full * 8192, tail * D)])

  @pl.when(wid == 1)
  def _():
    pltpu.sync_copy(tailj_hbm, obuf0.at[pl.ds(0, tail * D)])
    pltpu.sync_copy(obuf0.at[pl.ds(0, tail * D)],
                    wjo_hbm.at[pl.ds(n_full * 8192, tail * D)])

  for src, dst in ((wit_hbm, wio_hbm), (wjt_hbm, wjo_hbm)):

    def issue_in(blk, p):
      @pl.when(blk < n_full)
      def _():
        pltpu.async_copy(src.at[:, pl.ds(blk * 128, 128)], inbufs[p],
                         sem_in)

    def drain_in_full():
      pltpu.make_async_copy(src.at[:, pl.ds(0, 128)], inbuf0,
                            sem_in).wait()

    def drain_out_full():
      pltpu.make_async_copy(dst.at[pl.ds(0, 8192)], obuf0,
                            sem_out).wait()

    issue_in(wid, 0)

    def step(kk, p):
      blk = kk * NW + wid

      @pl.when((kk >= 2) & (blk < n_full))
      def _():
        drain_out_full()

      @pl.when(blk < n_full)
      def _():
        drain_in_full()
        issue_in(blk + NW, 1 - p)

        def rowf(d, _):
          for k8 in range(8):
            plsc.store_scatter(obufs[p], [pbase[k8] + d],
                               inbufs[p][d, pl.ds(16 * k8, 16)])
          return 0

        lax.fori_loop(0, D, rowf, 0)
        pltpu.async_copy(obufs[p], dst.at[pl.ds(blk * 8192, 8192)],
                         sem_out)

    def body(k, _):
      step(2 * k, 0)
      step(2 * k + 1, 1)
      return 0

    lax.fori_loop(0, (n_iters + 1) // 2, body, 0)

    drain_out_full()
    drain_out_full()
    plsc.subcore_barrier()


def _sc_transpose(wit, wjt, taili, tailj):
  V = wit.shape[1]
  mesh = plsc.VectorSubcoreMesh(core_axis_name="c", subcore_axis_name="s")
  f = pl.kernel(
      _transpose_body,
      out_type=[jax.ShapeDtypeStruct((V * D,), jnp.float32),
                jax.ShapeDtypeStruct((V * D,), jnp.float32)],
      mesh=mesh,
      compiler_params=pltpu.CompilerParams(needs_layout_passes=False,
                                           use_tc_tiling_on_sc=True),
      scratch_types=[
          pltpu.VMEM((D, 128), jnp.float32),
          pltpu.VMEM((D, 128), jnp.float32),
          pltpu.VMEM((8192,), jnp.float32),
          pltpu.VMEM((8192,), jnp.float32),
          pltpu.SemaphoreType.DMA,
          pltpu.SemaphoreType.DMA,
      ],
  )
  return f(wit, wjt, taili, tailj)


def _sc_body(i_hbm, j_hbm, neg_hbm, wi_hbm, wj_hbm, comb_hbm,
             iidx, jidx, nidx, wirows, wjrows, negrows, accscr, sbuf,
             sem0, sem1):
  B = i_hbm.shape[0]
  per_w = B // NW
  n_chunks = per_w // CHUNK
  nrows_per_chunk = CHUNK * N_NEG // 128
  dots_per_chunk = CHUNK * NDOT
  n_groups = dots_per_chunk // 16

  wid = lax.axis_index("s") * NC + lax.axis_index("c")
  base = wid * per_w

  pltpu.sync_copy(i_hbm.at[pl.ds(base, per_w)], iidx)
  pltpu.sync_copy(j_hbm.at[pl.ds(base, per_w)], jidx)
  nrow0 = wid * (per_w * N_NEG // 128)
  pltpu.sync_copy(neg_hbm.at[pl.ds(nrow0, per_w * N_NEG // 128)], nidx)

  sems = (sem0, sem1)
  lanes = lax.iota(jnp.int32, 16)

  def issue(c, p):
    sem = sems[p]
    descs = [
        pltpu.async_copy(wi_hbm.at[iidx.at[pl.ds(c * CHUNK, CHUNK)]],
                         wirows.at[p], sem),
        pltpu.async_copy(wj_hbm.at[jidx.at[pl.ds(c * CHUNK, CHUNK)]],
                         wjrows.at[p], sem),
    ]
    for r in range(nrows_per_chunk):
      descs.append(
          pltpu.async_copy(wj_hbm.at[nidx.at[c * nrows_per_chunk + r]],
                           negrows.at[p].at[pl.ds(r * 128, 128)], sem))
    return descs

  def compute(c, p):
    def item(b, _):
      wiv = [wirows[p, b, pl.ds(k * 16, 16)] for k in range(4)]
      acc = wiv[0] * wjrows[p, b, pl.ds(0, 16)]
      for k in range(1, 4):
        acc = acc + wiv[k] * wjrows[p, b, pl.ds(k * 16, 16)]
      accscr[pl.ds(b * NDOT * 16, 16)] = acc
      for n in range(N_NEG):
        r = b * N_NEG + n
        nacc = wiv[0] * negrows[p, r, pl.ds(0, 16)]
        for k in range(1, 4):
          nacc = nacc + wiv[k] * negrows[p, r, pl.ds(k * 16, 16)]
        accscr[pl.ds((b * NDOT + 1 + n) * 16, 16)] = nacc
      return 0

    lax.fori_loop(0, CHUNK, item, 0)

    def reduce_group(g, _):
      rows = (g * 16 + lanes) * 16
      red = plsc.load_gather(accscr, [rows])
      for i in range(1, 16):
        red = red + plsc.load_gather(accscr, [rows + i])
      sbuf[pl.ds(c * dots_per_chunk + g * 16, 16)] = red
      return 0

    lax.fori_loop(0, n_groups, reduce_group, 0)

  descs = issue(0, 0)
  for c in range(n_chunks):
    p = c & 1
    nxt = issue(c + 1, 1 - p) if c + 1 < n_chunks else []
    for d in descs:
      d.wait()
    compute(c, p)
    descs = nxt

  pltpu.sync_copy(sbuf, comb_hbm.at[pl.ds(base * NDOT, per_w * NDOT)])


def _sc_scores(i_idx, j_idx, neg2d, wi, wj):
  B = i_idx.shape[0]
  per_w = B // NW
  mesh = plsc.VectorSubcoreMesh(core_axis_name="c", subcore_axis_name="s")
  f = pl.kernel(
      _sc_body,
      out_type=jax.ShapeDtypeStruct((B * NDOT,), jnp.float32),
      mesh=mesh,
      compiler_params=pltpu.CompilerParams(needs_layout_passes=False,
                                           use_tc_tiling_on_sc=False),
      scratch_types=[
          pltpu.VMEM((per_w,), jnp.int32),
          pltpu.VMEM((per_w,), jnp.int32),
          pltpu.VMEM((per_w * N_NEG // 128, 128), jnp.int32),
          pltpu.VMEM((2, CHUNK, D), jnp.float32),
          pltpu.VMEM((2, CHUNK, D), jnp.float32),
          pltpu.VMEM((2, CHUNK * N_NEG, D), jnp.float32),
          pltpu.VMEM((CHUNK * NDOT * 16,), jnp.float32),
          pltpu.VMEM((per_w * NDOT,), jnp.float32),
          pltpu.SemaphoreType.DMA,
          pltpu.SemaphoreType.DMA,
      ],
  )
  return f(i_idx, j_idx, neg2d, wi, wj)


def _tc_loss_body(comb_ref, out_ref):
  rows, cols = comb_ref.shape
  flat = (lax.broadcasted_iota(jnp.int32, (rows, cols), 0) * cols
          + lax.broadcasted_iota(jnp.int32, (rows, cols), 1))
  is_pos = (flat % NDOT) == 0
  s = jnp.clip(comb_ref[...], -10.0, 10.0)
  x = jnp.where(is_pos, -s, s)
  loss = jnp.log1p(jnp.exp(x))
  out_ref[0, 0] = jnp.sum(loss) / (rows * cols // NDOT)


def _tc_loss(comb2d):
  out = pl.pallas_call(
      _tc_loss_body,
      out_shape=jax.ShapeDtypeStruct((1, 1), jnp.float32),
      out_specs=pl.BlockSpec(memory_space=pltpu.SMEM),
  )(comb2d)
  return out[0, 0]


def kernel(i_indices, j_indices, neg_indices, wi, wj):
  B = i_indices.shape[0]
  V = wi.shape[0]
  neg2d = neg_indices.reshape(B * N_NEG // 128, 128)
  tail = V - (V // 128) * 128
  taili = wi[V - tail:, :].reshape(tail * D)
  tailj = wj[V - tail:, :].reshape(tail * D)
  wi_lin, wj_lin = _sc_transpose(wi.T, wj.T, taili, tailj)
  comb = _sc_scores(i_indices, j_indices, neg2d,
                    wi_lin.reshape(V, D), wj_lin.reshape(V, D))
  return _tc_loss(comb.reshape(B * NDOT // 128, 128))


# speedup vs baseline: 3.1708x; 3.1708x over previous
import jax
import jax.numpy as jnp
from jax import lax
from jax.experimental import pallas as pl
from jax.experimental.pallas import tpu as pltpu
from jax.experimental.pallas import tpu_sc as plsc

D = 64
N_NEG = 20
NDOT = N_NEG + 1
NC, NS = 2, 16
NW = NC * NS
CHUNK = 32
TRC = 2048


def _tc_transpose_body(wit_ref, wjt_ref, wio_ref, wjo_ref):
  H = TRC // 2
  for src, dst in ((wit_ref, wio_ref), (wjt_ref, wjo_ref)):
    dst[:, 0:D] = src[:, 0:H].T
    dst[:, D:2 * D] = src[:, H:TRC].T


def _tc_transpose(wit, wjt):
  V = wit.shape[1]
  grid = (V + TRC - 1) // TRC
  nrows = grid * TRC * D // 128
  return pl.pallas_call(
      _tc_transpose_body,
      grid=(grid,),
      in_specs=[pl.BlockSpec((D, TRC), lambda i: (0, i)),
                pl.BlockSpec((D, TRC), lambda i: (0, i))],
      out_specs=[pl.BlockSpec((TRC // 2, 128), lambda i: (i, 0)),
                 pl.BlockSpec((TRC // 2, 128), lambda i: (i, 0))],
      out_shape=[jax.ShapeDtypeStruct((nrows, 128), jnp.float32),
                 jax.ShapeDtypeStruct((nrows, 128), jnp.float32)],
  )(wit, wjt)


def _sc_body(i_hbm, j_hbm, neg_hbm, wi_hbm, wj_hbm, comb_hbm,
             iidx, jidx, nidx, wirows, wjrows, negrows, accscr, sbuf,
             sem0, sem1):
  B = i_hbm.shape[0]
  per_w = B // NW
  n_chunks = per_w // CHUNK
  nrows_per_chunk = CHUNK * N_NEG // 128
  dots_per_chunk = CHUNK * NDOT
  n_groups = dots_per_chunk // 16

  wid = lax.axis_index("s") * NC + lax.axis_index("c")
  base = wid * per_w

  pltpu.sync_copy(i_hbm.at[pl.ds(base, per_w)], iidx)
  pltpu.sync_copy(j_hbm.at[pl.ds(base, per_w)], jidx)
  nrow0 = wid * (per_w * N_NEG // 128)
  pltpu.sync_copy(neg_hbm.at[pl.ds(nrow0, per_w * N_NEG // 128)], nidx)

  H = TRC // 2

  hshift = H.bit_length() - 1

  def remap(v):
    return (v & ~(TRC - 1)) | ((v & (H - 1)) << 1) | ((v >> hshift) & 1)

  def rloop(t, _):
    iidx[pl.ds(t * 16, 16)] = remap(iidx[pl.ds(t * 16, 16)])
    jidx[pl.ds(t * 16, 16)] = remap(jidx[pl.ds(t * 16, 16)])
    return 0

  lax.fori_loop(0, per_w // 16, rloop, 0)

  n_nrows = per_w * N_NEG // 128

  def nloop(t, _):
    row = t // 8
    c0 = (t % 8) * 16
    nidx[row, pl.ds(c0, 16)] = remap(nidx[row, pl.ds(c0, 16)])
    return 0

  lax.fori_loop(0, n_nrows * 8, nloop, 0)

  sems = (sem0, sem1)
  lanes = lax.iota(jnp.int32, 16)

  def issue(c, p):
    sem = sems[p]
    descs = [
        pltpu.async_copy(wi_hbm.at[iidx.at[pl.ds(c * CHUNK, CHUNK)]],
                         wirows.at[p], sem),
        pltpu.async_copy(wj_hbm.at[jidx.at[pl.ds(c * CHUNK, CHUNK)]],
                         wjrows.at[p], sem),
    ]
    for r in range(nrows_per_chunk):
      descs.append(
          pltpu.async_copy(wj_hbm.at[nidx.at[c * nrows_per_chunk + r]],
                           negrows.at[p].at[pl.ds(r * 128, 128)], sem))
    return descs

  def compute(c, p):
    def item(b, _):
      wiv = [wirows[p, b, pl.ds(k * 16, 16)] for k in range(4)]
      acc = wiv[0] * wjrows[p, b, pl.ds(0, 16)]
      for k in range(1, 4):
        acc = acc + wiv[k] * wjrows[p, b, pl.ds(k * 16, 16)]
      accscr[pl.ds(b * NDOT * 16, 16)] = acc
      for n in range(N_NEG):
        r = b * N_NEG + n
        nacc = wiv[0] * negrows[p, r, pl.ds(0, 16)]
        for k in range(1, 4):
          nacc = nacc + wiv[k] * negrows[p, r, pl.ds(k * 16, 16)]
        accscr[pl.ds((b * NDOT + 1 + n) * 16, 16)] = nacc
      return 0

    lax.fori_loop(0, CHUNK, item, 0)

    def reduce_group(g, _):
      rows = (g * 16 + lanes) * 16
      red = plsc.load_gather(accscr, [rows])
      for i in range(1, 16):
        red = red + plsc.load_gather(accscr, [rows + i])
      sbuf[pl.ds(c * dots_per_chunk + g * 16, 16)] = red
      return 0

    lax.fori_loop(0, n_groups, reduce_group, 0)

  descs = issue(0, 0)
  for c in range(n_chunks):
    p = c & 1
    nxt = issue(c + 1, 1 - p) if c + 1 < n_chunks else []
    for d in descs:
      d.wait()
    compute(c, p)
    descs = nxt

  pltpu.sync_copy(sbuf, comb_hbm.at[pl.ds(base * NDOT, per_w * NDOT)])


def _sc_scores(i_idx, j_idx, neg2d, wi, wj):
  B = i_idx.shape[0]
  per_w = B // NW
  mesh = plsc.VectorSubcoreMesh(core_axis_name="c", subcore_axis_name="s")
  f = pl.kernel(
      _sc_body,
      out_type=jax.ShapeDtypeStruct((B * NDOT,), jnp.float32),
      mesh=mesh,
      compiler_params=pltpu.CompilerParams(needs_layout_passes=False,
                                           use_tc_tiling_on_sc=False),
      scratch_types=[
          pltpu.VMEM((per_w,), jnp.int32),
          pltpu.VMEM((per_w,), jnp.int32),
          pltpu.VMEM((per_w * N_NEG // 128, 128), jnp.int32),
          pltpu.VMEM((2, CHUNK, D), jnp.float32),
          pltpu.VMEM((2, CHUNK, D), jnp.float32),
          pltpu.VMEM((2, CHUNK * N_NEG, D), jnp.float32),
          pltpu.VMEM((CHUNK * NDOT * 16,), jnp.float32),
          pltpu.VMEM((per_w * NDOT,), jnp.float32),
          pltpu.SemaphoreType.DMA,
          pltpu.SemaphoreType.DMA,
      ],
  )
  return f(i_idx, j_idx, neg2d, wi, wj)


def _tc_loss_body(comb_ref, out_ref):
  rows, cols = comb_ref.shape
  flat = (lax.broadcasted_iota(jnp.int32, (rows, cols), 0) * cols
          + lax.broadcasted_iota(jnp.int32, (rows, cols), 1))
  is_pos = (flat % NDOT) == 0
  s = jnp.clip(comb_ref[...], -10.0, 10.0)
  x = jnp.where(is_pos, -s, s)
  loss = jnp.log1p(jnp.exp(x))
  out_ref[0, 0] = jnp.sum(loss) / (rows * cols // NDOT)


def _tc_loss(comb2d):
  out = pl.pallas_call(
      _tc_loss_body,
      out_shape=jax.ShapeDtypeStruct((1, 1), jnp.float32),
      out_specs=pl.BlockSpec(memory_space=pltpu.SMEM),
  )(comb2d)
  return out[0, 0]


def kernel(i_indices, j_indices, neg_indices, wi, wj):
  B = i_indices.shape[0]
  V = wi.shape[0]
  VP = ((V + TRC - 1) // TRC) * TRC
  neg2d = neg_indices.reshape(B * N_NEG // 128, 128)
  wi_lin, wj_lin = _tc_transpose(wi.T, wj.T)
  comb = _sc_scores(i_indices, j_indices, neg2d,
                    wi_lin.reshape(VP, D), wj_lin.reshape(VP, D))
  return _tc_loss(comb.reshape(B * NDOT // 128, 128))


# speedup vs baseline: 3.8342x; 1.2092x over previous
import jax
import jax.numpy as jnp
from jax import lax
from jax.experimental import pallas as pl
from jax.experimental.pallas import tpu as pltpu
from jax.experimental.pallas import tpu_sc as plsc

D = 64
N_NEG = 20
NDOT = N_NEG + 1
NC, NS = 2, 16
NW = NC * NS
CHUNK = 32
TRC = 4096


def _tc_transpose_body(wit_ref, wjt_ref, wio_ref, wjo_ref):
  H = TRC // 2
  for src, dst in ((wit_ref, wio_ref), (wjt_ref, wjo_ref)):
    dst[:, 0:D] = src[:, 0:H].T
    dst[:, D:2 * D] = src[:, H:TRC].T


def _tc_transpose(wit, wjt):
  V = wit.shape[1]
  grid = (V + TRC - 1) // TRC
  nrows = grid * TRC * D // 128
  return pl.pallas_call(
      _tc_transpose_body,
      grid=(grid,),
      in_specs=[pl.BlockSpec((D, TRC), lambda i: (0, i)),
                pl.BlockSpec((D, TRC), lambda i: (0, i))],
      out_specs=[pl.BlockSpec((TRC // 2, 128), lambda i: (i, 0)),
                 pl.BlockSpec((TRC // 2, 128), lambda i: (i, 0))],
      out_shape=[jax.ShapeDtypeStruct((nrows, 128), jnp.float32),
                 jax.ShapeDtypeStruct((nrows, 128), jnp.float32)],
  )(wit, wjt)


def _sc_body(i_hbm, j_hbm, neg_hbm, wi_hbm, wj_hbm, comb_hbm,
             iidx, jidx, nidx, wirows, wjrows, negrows, accscr, sbuf,
             sem0, sem1):
  B = i_hbm.shape[0]
  per_w = B // NW
  n_chunks = per_w // CHUNK
  nrows_per_chunk = CHUNK * N_NEG // 128
  dots_per_chunk = CHUNK * NDOT
  n_groups = dots_per_chunk // 16

  wid = lax.axis_index("s") * NC + lax.axis_index("c")
  base = wid * per_w

  pltpu.sync_copy(i_hbm.at[pl.ds(base, per_w)], iidx)
  pltpu.sync_copy(j_hbm.at[pl.ds(base, per_w)], jidx)
  nrow0 = wid * (per_w * N_NEG // 128)
  pltpu.sync_copy(neg_hbm.at[pl.ds(nrow0, per_w * N_NEG // 128)], nidx)

  H = TRC // 2

  hshift = H.bit_length() - 1

  def remap(v):
    return (v & ~(TRC - 1)) | ((v & (H - 1)) << 1) | ((v >> hshift) & 1)

  def rloop(t, _):
    iidx[pl.ds(t * 16, 16)] = remap(iidx[pl.ds(t * 16, 16)])
    jidx[pl.ds(t * 16, 16)] = remap(jidx[pl.ds(t * 16, 16)])
    return 0

  lax.fori_loop(0, per_w // 16, rloop, 0)

  n_nrows = per_w * N_NEG // 128

  def nloop(t, _):
    row = t // 8
    c0 = (t % 8) * 16
    nidx[row, pl.ds(c0, 16)] = remap(nidx[row, pl.ds(c0, 16)])
    return 0

  lax.fori_loop(0, n_nrows * 8, nloop, 0)

  sems = (sem0, sem1)
  lanes = lax.iota(jnp.int32, 16)

  def issue(c, p):
    sem = sems[p]
    descs = [
        pltpu.async_copy(wi_hbm.at[iidx.at[pl.ds(c * CHUNK, CHUNK)]],
                         wirows.at[p], sem),
        pltpu.async_copy(wj_hbm.at[jidx.at[pl.ds(c * CHUNK, CHUNK)]],
                         wjrows.at[p], sem),
    ]
    for r in range(nrows_per_chunk):
      descs.append(
          pltpu.async_copy(wj_hbm.at[nidx.at[c * nrows_per_chunk + r]],
                           negrows.at[p].at[pl.ds(r * 128, 128)], sem))
    return descs

  def compute(c, p):
    def item(b, _):
      wiv = [wirows[p, b, pl.ds(k * 16, 16)] for k in range(4)]
      acc = wiv[0] * wjrows[p, b, pl.ds(0, 16)]
      for k in range(1, 4):
        acc = acc + wiv[k] * wjrows[p, b, pl.ds(k * 16, 16)]
      accscr[pl.ds(b * NDOT * 16, 16)] = acc
      for n in range(N_NEG):
        r = b * N_NEG + n
        nacc = wiv[0] * negrows[p, r, pl.ds(0, 16)]
        for k in range(1, 4):
          nacc = nacc + wiv[k] * negrows[p, r, pl.ds(k * 16, 16)]
        accscr[pl.ds((b * NDOT + 1 + n) * 16, 16)] = nacc
      return 0

    lax.fori_loop(0, CHUNK, item, 0)

    def reduce_group(g, _):
      rows = (g * 16 + lanes) * 16
      red = plsc.load_gather(accscr, [rows])
      for i in range(1, 16):
        red = red + plsc.load_gather(accscr, [rows + i])
      sbuf[pl.ds(c * dots_per_chunk + g * 16, 16)] = red
      return 0

    lax.fori_loop(0, n_groups, reduce_group, 0)

  descs = issue(0, 0)
  for c in range(n_chunks):
    p = c & 1
    nxt = issue(c + 1, 1 - p) if c + 1 < n_chunks else []
    for d in descs:
      d.wait()
    compute(c, p)
    descs = nxt

  pltpu.sync_copy(sbuf, comb_hbm.at[pl.ds(base * NDOT, per_w * NDOT)])


def _sc_scores(i_idx, j_idx, neg2d, wi, wj):
  B = i_idx.shape[0]
  per_w = B // NW
  mesh = plsc.VectorSubcoreMesh(core_axis_name="c", subcore_axis_name="s")
  f = pl.kernel(
      _sc_body,
      out_type=jax.ShapeDtypeStruct((B * NDOT,), jnp.float32),
      mesh=mesh,
      compiler_params=pltpu.CompilerParams(needs_layout_passes=False,
                                           use_tc_tiling_on_sc=False),
      scratch_types=[
          pltpu.VMEM((per_w,), jnp.int32),
          pltpu.VMEM((per_w,), jnp.int32),
          pltpu.VMEM((per_w * N_NEG // 128, 128), jnp.int32),
          pltpu.VMEM((2, CHUNK, D), jnp.float32),
          pltpu.VMEM((2, CHUNK, D), jnp.float32),
          pltpu.VMEM((2, CHUNK * N_NEG, D), jnp.float32),
          pltpu.VMEM((CHUNK * NDOT * 16,), jnp.float32),
          pltpu.VMEM((per_w * NDOT,), jnp.float32),
          pltpu.SemaphoreType.DMA,
          pltpu.SemaphoreType.DMA,
      ],
  )
  return f(i_idx, j_idx, neg2d, wi, wj)


def _tc_loss_body(comb_ref, out_ref):
  rows, cols = comb_ref.shape
  flat = (lax.broadcasted_iota(jnp.int32, (rows, cols), 0) * cols
          + lax.broadcasted_iota(jnp.int32, (rows, cols), 1))
  is_pos = (flat % NDOT) == 0
  s = jnp.clip(comb_ref[...], -10.0, 10.0)
  x = jnp.where(is_pos, -s, s)
  loss = jnp.log1p(jnp.exp(x))
  out_ref[0, 0] = jnp.sum(loss) / (rows * cols // NDOT)


def _tc_loss(comb2d):
  out = pl.pallas_call(
      _tc_loss_body,
      out_shape=jax.ShapeDtypeStruct((1, 1), jnp.float32),
      out_specs=pl.BlockSpec(memory_space=pltpu.SMEM),
  )(comb2d)
  return out[0, 0]


def kernel(i_indices, j_indices, neg_indices, wi, wj):
  B = i_indices.shape[0]
  V = wi.shape[0]
  VP = ((V + TRC - 1) // TRC) * TRC
  neg2d = neg_indices.reshape(B * N_NEG // 128, 128)
  wi_lin, wj_lin = _tc_transpose(wi.T, wj.T)
  comb = _sc_scores(i_indices, j_indices, neg2d,
                    wi_lin.reshape(VP, D), wj_lin.reshape(VP, D))
  return _tc_loss(comb.reshape(B * NDOT // 128, 128))


# speedup vs baseline: 4.3488x; 1.1342x over previous
import jax
import jax.numpy as jnp
from jax import lax
from jax.experimental import pallas as pl
from jax.experimental.pallas import tpu as pltpu
from jax.experimental.pallas import tpu_sc as plsc

D = 64
N_NEG = 20
NDOT = N_NEG + 1
NC, NS = 2, 16
NW = NC * NS
CHUNK = 32
TRC = 8192


def _tc_transpose_body(wit_ref, wjt_ref, wio_ref, wjo_ref):
  H = TRC // 2
  for src, dst in ((wit_ref, wio_ref), (wjt_ref, wjo_ref)):
    dst[:, 0:D] = src[:, 0:H].T
    dst[:, D:2 * D] = src[:, H:TRC].T


def _tc_transpose(wit, wjt):
  V = wit.shape[1]
  grid = (V + TRC - 1) // TRC
  nrows = grid * TRC * D // 128
  return pl.pallas_call(
      _tc_transpose_body,
      grid=(grid,),
      in_specs=[pl.BlockSpec((D, TRC), lambda i: (0, i)),
                pl.BlockSpec((D, TRC), lambda i: (0, i))],
      out_specs=[pl.BlockSpec((TRC // 2, 128), lambda i: (i, 0)),
                 pl.BlockSpec((TRC // 2, 128), lambda i: (i, 0))],
      out_shape=[jax.ShapeDtypeStruct((nrows, 128), jnp.float32),
                 jax.ShapeDtypeStruct((nrows, 128), jnp.float32)],
  )(wit, wjt)


def _sc_body(i_hbm, j_hbm, neg_hbm, wi_hbm, wj_hbm, comb_hbm,
             iidx, jidx, nidx, wirows, wjrows, negrows, accscr, sbuf,
             sem0, sem1):
  B = i_hbm.shape[0]
  per_w = B // NW
  n_chunks = per_w // CHUNK
  nrows_per_chunk = CHUNK * N_NEG // 128
  dots_per_chunk = CHUNK * NDOT
  n_groups = dots_per_chunk // 16

  wid = lax.axis_index("s") * NC + lax.axis_index("c")
  base = wid * per_w

  pltpu.sync_copy(i_hbm.at[pl.ds(base, per_w)], iidx)
  pltpu.sync_copy(j_hbm.at[pl.ds(base, per_w)], jidx)
  nrow0 = wid * (per_w * N_NEG // 128)
  pltpu.sync_copy(neg_hbm.at[pl.ds(nrow0, per_w * N_NEG // 128)], nidx)

  H = TRC // 2

  hshift = H.bit_length() - 1

  def remap(v):
    return (v & ~(TRC - 1)) | ((v & (H - 1)) << 1) | ((v >> hshift) & 1)

  def rloop(t, _):
    iidx[pl.ds(t * 16, 16)] = remap(iidx[pl.ds(t * 16, 16)])
    jidx[pl.ds(t * 16, 16)] = remap(jidx[pl.ds(t * 16, 16)])
    return 0

  lax.fori_loop(0, per_w // 16, rloop, 0)

  n_nrows = per_w * N_NEG // 128

  def nloop(t, _):
    row = t // 8
    c0 = (t % 8) * 16
    nidx[row, pl.ds(c0, 16)] = remap(nidx[row, pl.ds(c0, 16)])
    return 0

  lax.fori_loop(0, n_nrows * 8, nloop, 0)

  sems = (sem0, sem1)
  lanes = lax.iota(jnp.int32, 16)

  def issue(c, p):
    sem = sems[p]
    descs = [
        pltpu.async_copy(wi_hbm.at[iidx.at[pl.ds(c * CHUNK, CHUNK)]],
                         wirows.at[p], sem),
        pltpu.async_copy(wj_hbm.at[jidx.at[pl.ds(c * CHUNK, CHUNK)]],
                         wjrows.at[p], sem),
    ]
    for r in range(nrows_per_chunk):
      descs.append(
          pltpu.async_copy(wj_hbm.at[nidx.at[c * nrows_per_chunk + r]],
                           negrows.at[p].at[pl.ds(r * 128, 128)], sem))
    return descs

  def compute(c, p):
    def item(b, _):
      wiv = [wirows[p, b, pl.ds(k * 16, 16)] for k in range(4)]
      acc = wiv[0] * wjrows[p, b, pl.ds(0, 16)]
      for k in range(1, 4):
        acc = acc + wiv[k] * wjrows[p, b, pl.ds(k * 16, 16)]
      accscr[pl.ds(b * NDOT * 16, 16)] = acc
      for n in range(N_NEG):
        r = b * N_NEG + n
        nacc = wiv[0] * negrows[p, r, pl.ds(0, 16)]
        for k in range(1, 4):
          nacc = nacc + wiv[k] * negrows[p, r, pl.ds(k * 16, 16)]
        accscr[pl.ds((b * NDOT + 1 + n) * 16, 16)] = nacc
      return 0

    lax.fori_loop(0, CHUNK, item, 0)

    def reduce_group(g, _):
      rows = (g * 16 + lanes) * 16
      red = plsc.load_gather(accscr, [rows])
      for i in range(1, 16):
        red = red + plsc.load_gather(accscr, [rows + i])
      sbuf[pl.ds(c * dots_per_chunk + g * 16, 16)] = red
      return 0

    lax.fori_loop(0, n_groups, reduce_group, 0)

  descs = issue(0, 0)
  for c in range(n_chunks):
    p = c & 1
    nxt = issue(c + 1, 1 - p) if c + 1 < n_chunks else []
    for d in descs:
      d.wait()
    compute(c, p)
    descs = nxt

  pltpu.sync_copy(sbuf, comb_hbm.at[pl.ds(base * NDOT, per_w * NDOT)])


def _sc_scores(i_idx, j_idx, neg2d, wi, wj):
  B = i_idx.shape[0]
  per_w = B // NW
  mesh = plsc.VectorSubcoreMesh(core_axis_name="c", subcore_axis_name="s")
  f = pl.kernel(
      _sc_body,
      out_type=jax.ShapeDtypeStruct((B * NDOT,), jnp.float32),
      mesh=mesh,
      compiler_params=pltpu.CompilerParams(needs_layout_passes=False,
                                           use_tc_tiling_on_sc=False),
      scratch_types=[
          pltpu.VMEM((per_w,), jnp.int32),
          pltpu.VMEM((per_w,), jnp.int32),
          pltpu.VMEM((per_w * N_NEG // 128, 128), jnp.int32),
          pltpu.VMEM((2, CHUNK, D), jnp.float32),
          pltpu.VMEM((2, CHUNK, D), jnp.float32),
          pltpu.VMEM((2, CHUNK * N_NEG, D), jnp.float32),
          pltpu.VMEM((CHUNK * NDOT * 16,), jnp.float32),
          pltpu.VMEM((per_w * NDOT,), jnp.float32),
          pltpu.SemaphoreType.DMA,
          pltpu.SemaphoreType.DMA,
      ],
  )
  return f(i_idx, j_idx, neg2d, wi, wj)


def _tc_loss_body(comb_ref, out_ref):
  rows, cols = comb_ref.shape
  flat = (lax.broadcasted_iota(jnp.int32, (rows, cols), 0) * cols
          + lax.broadcasted_iota(jnp.int32, (rows, cols), 1))
  is_pos = (flat % NDOT) == 0
  s = jnp.clip(comb_ref[...], -10.0, 10.0)
  x = jnp.where(is_pos, -s, s)
  loss = jnp.log1p(jnp.exp(x))
  out_ref[0, 0] = jnp.sum(loss) / (rows * cols // NDOT)


def _tc_loss(comb2d):
  out = pl.pallas_call(
      _tc_loss_body,
      out_shape=jax.ShapeDtypeStruct((1, 1), jnp.float32),
      out_specs=pl.BlockSpec(memory_space=pltpu.SMEM),
  )(comb2d)
  return out[0, 0]


def kernel(i_indices, j_indices, neg_indices, wi, wj):
  B = i_indices.shape[0]
  V = wi.shape[0]
  VP = ((V + TRC - 1) // TRC) * TRC
  neg2d = neg_indices.reshape(B * N_NEG // 128, 128)
  wi_lin, wj_lin = _tc_transpose(wi.T, wj.T)
  comb = _sc_scores(i_indices, j_indices, neg2d,
                    wi_lin.reshape(VP, D), wj_lin.reshape(VP, D))
  return _tc_loss(comb.reshape(B * NDOT // 128, 128))


# speedup vs baseline: 4.4101x; 1.0141x over previous
import jax
import jax.numpy as jnp
from jax import lax
from jax.experimental import pallas as pl
from jax.experimental.pallas import tpu as pltpu
from jax.experimental.pallas import tpu_sc as plsc

D = 64
N_NEG = 20
NDOT = N_NEG + 1
NC, NS = 2, 16
NW = NC * NS
CHUNK = 32
TRC = 16384


def _tc_transpose_body(wit_ref, wjt_ref, wio_ref, wjo_ref):
  H = TRC // 2
  for src, dst in ((wit_ref, wio_ref), (wjt_ref, wjo_ref)):
    dst[:, 0:D] = src[:, 0:H].T
    dst[:, D:2 * D] = src[:, H:TRC].T


def _tc_transpose(wit, wjt):
  V = wit.shape[1]
  grid = (V + TRC - 1) // TRC
  nrows = grid * TRC * D // 128
  return pl.pallas_call(
      _tc_transpose_body,
      grid=(grid,),
      in_specs=[pl.BlockSpec((D, TRC), lambda i: (0, i)),
                pl.BlockSpec((D, TRC), lambda i: (0, i))],
      out_specs=[pl.BlockSpec((TRC // 2, 128), lambda i: (i, 0)),
                 pl.BlockSpec((TRC // 2, 128), lambda i: (i, 0))],
      out_shape=[jax.ShapeDtypeStruct((nrows, 128), jnp.float32),
                 jax.ShapeDtypeStruct((nrows, 128), jnp.float32)],
  )(wit, wjt)


def _sc_body(i_hbm, j_hbm, neg_hbm, wi_hbm, wj_hbm, comb_hbm,
             iidx, jidx, nidx, wirows, wjrows, negrows, accscr, sbuf,
             sem0, sem1):
  B = i_hbm.shape[0]
  per_w = B // NW
  n_chunks = per_w // CHUNK
  nrows_per_chunk = CHUNK * N_NEG // 128
  dots_per_chunk = CHUNK * NDOT
  n_groups = dots_per_chunk // 16

  wid = lax.axis_index("s") * NC + lax.axis_index("c")
  base = wid * per_w

  pltpu.sync_copy(i_hbm.at[pl.ds(base, per_w)], iidx)
  pltpu.sync_copy(j_hbm.at[pl.ds(base, per_w)], jidx)
  nrow0 = wid * (per_w * N_NEG // 128)
  pltpu.sync_copy(neg_hbm.at[pl.ds(nrow0, per_w * N_NEG // 128)], nidx)

  H = TRC // 2

  hshift = H.bit_length() - 1

  def remap(v):
    return (v & ~(TRC - 1)) | ((v & (H - 1)) << 1) | ((v >> hshift) & 1)

  def rloop(t, _):
    iidx[pl.ds(t * 16, 16)] = remap(iidx[pl.ds(t * 16, 16)])
    jidx[pl.ds(t * 16, 16)] = remap(jidx[pl.ds(t * 16, 16)])
    return 0

  lax.fori_loop(0, per_w // 16, rloop, 0)

  n_nrows = per_w * N_NEG // 128

  def nloop(t, _):
    row = t // 8
    c0 = (t % 8) * 16
    nidx[row, pl.ds(c0, 16)] = remap(nidx[row, pl.ds(c0, 16)])
    return 0

  lax.fori_loop(0, n_nrows * 8, nloop, 0)

  sems = (sem0, sem1)
  lanes = lax.iota(jnp.int32, 16)

  def issue(c, p):
    sem = sems[p]
    descs = [
        pltpu.async_copy(wi_hbm.at[iidx.at[pl.ds(c * CHUNK, CHUNK)]],
                         wirows.at[p], sem),
        pltpu.async_copy(wj_hbm.at[jidx.at[pl.ds(c * CHUNK, CHUNK)]],
                         wjrows.at[p], sem),
    ]
    for r in range(nrows_per_chunk):
      descs.append(
          pltpu.async_copy(wj_hbm.at[nidx.at[c * nrows_per_chunk + r]],
                           negrows.at[p].at[pl.ds(r * 128, 128)], sem))
    return descs

  def compute(c, p):
    def item(b, _):
      wiv = [wirows[p, b, pl.ds(k * 16, 16)] for k in range(4)]
      acc = wiv[0] * wjrows[p, b, pl.ds(0, 16)]
      for k in range(1, 4):
        acc = acc + wiv[k] * wjrows[p, b, pl.ds(k * 16, 16)]
      accscr[pl.ds(b * NDOT * 16, 16)] = acc
      for n in range(N_NEG):
        r = b * N_NEG + n
        nacc = wiv[0] * negrows[p, r, pl.ds(0, 16)]
        for k in range(1, 4):
          nacc = nacc + wiv[k] * negrows[p, r, pl.ds(k * 16, 16)]
        accscr[pl.ds((b * NDOT + 1 + n) * 16, 16)] = nacc
      return 0

    lax.fori_loop(0, CHUNK, item, 0)

    def reduce_group(g, _):
      rows = (g * 16 + lanes) * 16
      red = plsc.load_gather(accscr, [rows])
      for i in range(1, 16):
        red = red + plsc.load_gather(accscr, [rows + i])
      sbuf[pl.ds(c * dots_per_chunk + g * 16, 16)] = red
      return 0

    lax.fori_loop(0, n_groups, reduce_group, 0)

  descs = issue(0, 0)
  for c in range(n_chunks):
    p = c & 1
    nxt = issue(c + 1, 1 - p) if c + 1 < n_chunks else []
    for d in descs:
      d.wait()
    compute(c, p)
    descs = nxt

  pltpu.sync_copy(sbuf, comb_hbm.at[pl.ds(base * NDOT, per_w * NDOT)])


def _sc_scores(i_idx, j_idx, neg2d, wi, wj):
  B = i_idx.shape[0]
  per_w = B // NW
  mesh = plsc.VectorSubcoreMesh(core_axis_name="c", subcore_axis_name="s")
  f = pl.kernel(
      _sc_body,
      out_type=jax.ShapeDtypeStruct((B * NDOT,), jnp.float32),
      mesh=mesh,
      compiler_params=pltpu.CompilerParams(needs_layout_passes=False,
                                           use_tc_tiling_on_sc=False),
      scratch_types=[
          pltpu.VMEM((per_w,), jnp.int32),
          pltpu.VMEM((per_w,), jnp.int32),
          pltpu.VMEM((per_w * N_NEG // 128, 128), jnp.int32),
          pltpu.VMEM((2, CHUNK, D), jnp.float32),
          pltpu.VMEM((2, CHUNK, D), jnp.float32),
          pltpu.VMEM((2, CHUNK * N_NEG, D), jnp.float32),
          pltpu.VMEM((CHUNK * NDOT * 16,), jnp.float32),
          pltpu.VMEM((per_w * NDOT,), jnp.float32),
          pltpu.SemaphoreType.DMA,
          pltpu.SemaphoreType.DMA,
      ],
  )
  return f(i_idx, j_idx, neg2d, wi, wj)


def _tc_loss_body(comb_ref, out_ref):
  rows, cols = comb_ref.shape
  flat = (lax.broadcasted_iota(jnp.int32, (rows, cols), 0) * cols
          + lax.broadcasted_iota(jnp.int32, (rows, cols), 1))
  is_pos = (flat % NDOT) == 0
  s = jnp.clip(comb_ref[...], -10.0, 10.0)
  x = jnp.where(is_pos, -s, s)
  loss = jnp.log1p(jnp.exp(x))
  out_ref[0, 0] = jnp.sum(loss) / (rows * cols // NDOT)


def _tc_loss(comb2d):
  out = pl.pallas_call(
      _tc_loss_body,
      out_shape=jax.ShapeDtypeStruct((1, 1), jnp.float32),
      out_specs=pl.BlockSpec(memory_space=pltpu.SMEM),
  )(comb2d)
  return out[0, 0]


def kernel(i_indices, j_indices, neg_indices, wi, wj):
  B = i_indices.shape[0]
  V = wi.shape[0]
  VP = ((V + TRC - 1) // TRC) * TRC
  neg2d = neg_indices.reshape(B * N_NEG // 128, 128)
  wi_lin, wj_lin = _tc_transpose(wi.T, wj.T)
  comb = _sc_scores(i_indices, j_indices, neg2d,
                    wi_lin.reshape(VP, D), wj_lin.reshape(VP, D))
  return _tc_loss(comb.reshape(B * NDOT // 128, 128))
